# Initial kernel scaffold; baseline (speedup 1.0000x reference)
import jax
import jax.numpy as jnp
from jax.experimental import pallas as pl


def kernel(qid_list, input_processed_assessment_list, input_finished_time_list, part_list, W_question, W_position, W_assessment, W_finished_time, W_part):
    raise NotImplementedError("write your pallas kernel here")


# speedup vs baseline: 11.6996x; 11.6996x over previous
import functools

import jax
import jax.numpy as jnp
from jax import lax
from jax.experimental import pallas as pl
from jax.experimental.pallas import tpu as pltpu
from jax.experimental.pallas import tpu_sc as plsc

B, L, D = 4096, 200, 128
VQ, VA, VT, VP = 100000, 4, 301, 8
N = B * L
VS = VT * VA * VP

NC, NS = 2, 16
NW = NC * NS
TPW = N // NW
C = 128
NCHUNK = TPW // C


def _add_body(a_ref, b_ref, o_ref):
    o_ref[...] = a_ref[...] + b_ref[...]


def _table_add(a, b):
    rows = a.shape[0]
    br = 1000
    return pl.pallas_call(
        _add_body,
        grid=(rows // br,),
        in_specs=[pl.BlockSpec((br, D), lambda i: (i, 0))] * 2,
        out_specs=pl.BlockSpec((br, D), lambda i: (i, 0)),
        out_shape=jax.ShapeDtypeStruct((rows, D), jnp.float32),
    )(a, b)


_SBR = 2408


def _small_body(wt_ref, wa_ref, wp_ref, o_ref):
    r0 = pl.program_id(0) * _SBR
    i_t = lax.broadcasted_iota(jnp.int32, (_SBR, VT), 0) + r0
    j_t = lax.broadcasted_iota(jnp.int32, (_SBR, VT), 1)
    oh_t = ((i_t // (VA * VP)) == j_t).astype(jnp.float32)
    i_a = lax.broadcasted_iota(jnp.int32, (_SBR, VA), 0) + r0
    j_a = lax.broadcasted_iota(jnp.int32, (_SBR, VA), 1)
    oh_a = (((i_a // VP) % VA) == j_a).astype(jnp.float32)
    i_p = lax.broadcasted_iota(jnp.int32, (_SBR, VP), 0) + r0
    j_p = lax.broadcasted_iota(jnp.int32, (_SBR, VP), 1)
    oh_p = ((i_p % VP) == j_p).astype(jnp.float32)
    acc = jnp.dot(oh_t, wt_ref[...], preferred_element_type=jnp.float32)
    acc += jnp.dot(oh_a, wa_ref[...], preferred_element_type=jnp.float32)
    acc += jnp.dot(oh_p, wp_ref[...], preferred_element_type=jnp.float32)
    o_ref[...] = acc


def _small_table(wt, wa, wp):
    return pl.pallas_call(
        _small_body,
        grid=(VS // _SBR,),
        in_specs=[
            pl.BlockSpec((VT, D), lambda i: (0, 0)),
            pl.BlockSpec((VA, D), lambda i: (0, 0)),
            pl.BlockSpec((VP, D), lambda i: (0, 0)),
        ],
        out_specs=pl.BlockSpec((_SBR, D), lambda i: (i, 0)),
        out_shape=jax.ShapeDtypeStruct((VS, D), jnp.float32),
    )(wt, wa, wp)


def _sc_body(qid_hbm, t_hbm, a_hbm, p_hbm, wqp_hbm, wsmall_hbm, out_hbm,
             qid_v, t_v, a_v, p_v, s_v, rows_v, sem):
    cid = lax.axis_index("c")
    sid = lax.axis_index("s")
    wid = sid * NC + cid
    base = wid * TPW

    def chunk_body(g, carry):
        off = base + g * C
        pltpu.sync_copy(qid_hbm.at[pl.ds(off, C)], qid_v)
        pltpu.sync_copy(t_hbm.at[pl.ds(off, C)], t_v)
        pltpu.sync_copy(a_hbm.at[pl.ds(off, C)], a_v)
        pltpu.sync_copy(p_hbm.at[pl.ds(off, C)], p_v)
        for i in range(C // 16):
            sl = pl.ds(i * 16, 16)
            s_v[sl] = t_v[sl] * (VA * VP) + a_v[sl] * VP + p_v[sl]
        pltpu.async_copy(wqp_hbm.at[qid_v], rows_v, sem).wait()
        pltpu.async_copy(wsmall_hbm.at[s_v], rows_v, sem, add=True).wait()
        pltpu.sync_copy(rows_v, out_hbm.at[pl.ds(off, C)])
        return carry

    lax.fori_loop(0, NCHUNK, chunk_body, 0)


_sc_embed = functools.partial(
    pl.kernel,
    out_type=jax.ShapeDtypeStruct((N, D), jnp.float32),
    mesh=plsc.VectorSubcoreMesh(core_axis_name="c", subcore_axis_name="s"),
    scratch_types=[
        pltpu.VMEM((C,), jnp.int32),
        pltpu.VMEM((C,), jnp.int32),
        pltpu.VMEM((C,), jnp.int32),
        pltpu.VMEM((C,), jnp.int32),
        pltpu.VMEM((C,), jnp.int32),
        pltpu.VMEM((C, D), jnp.float32),
        pltpu.SemaphoreType.DMA,
    ],
)(_sc_body)


def kernel(qid_list, input_processed_assessment_list, input_finished_time_list,
           part_list, W_question, W_position, W_assessment, W_finished_time,
           W_part):
    qid = qid_list.reshape(N).astype(jnp.int32)
    a = input_processed_assessment_list.reshape(N).astype(jnp.int32)
    t = input_finished_time_list.reshape(N).astype(jnp.int32)
    p = part_list.reshape(N).astype(jnp.int32)
    wqp = _table_add(W_question, W_position)
    wsmall = _small_table(W_finished_time, W_assessment, W_part)
    out = _sc_embed(qid, t, a, p, wqp, wsmall)
    return out.reshape(B, L, D)


# speedup vs baseline: 25.0627x; 2.1422x over previous
import functools

import jax
import jax.numpy as jnp
from jax import lax
from jax.experimental import pallas as pl
from jax.experimental.pallas import tpu as pltpu
from jax.experimental.pallas import tpu_sc as plsc

B, L, D = 4096, 200, 128
VQ, VA, VT, VP = 100000, 4, 301, 8
N = B * L
VS = VT * VA * VP

NC, NS = 2, 16
NW = NC * NS
TPW = N // NW
C = 128
NCHUNK = TPW // C
NBUF = 4


def _add_body(a_ref, b_ref, o_ref):
    o_ref[...] = a_ref[...] + b_ref[...]


def _table_add(a, b):
    rows = a.shape[0]
    br = 1000
    return pl.pallas_call(
        _add_body,
        grid=(rows // br,),
        in_specs=[pl.BlockSpec((br, D), lambda i: (i, 0))] * 2,
        out_specs=pl.BlockSpec((br, D), lambda i: (i, 0)),
        out_shape=jax.ShapeDtypeStruct((rows, D), jnp.float32),
    )(a, b)


_SBR = 2408


def _small_body(wt_ref, wa_ref, wp_ref, o_ref):
    r0 = pl.program_id(0) * _SBR
    i_t = lax.broadcasted_iota(jnp.int32, (_SBR, VT), 0) + r0
    j_t = lax.broadcasted_iota(jnp.int32, (_SBR, VT), 1)
    oh_t = ((i_t // (VA * VP)) == j_t).astype(jnp.float32)
    i_a = lax.broadcasted_iota(jnp.int32, (_SBR, VA), 0) + r0
    j_a = lax.broadcasted_iota(jnp.int32, (_SBR, VA), 1)
    oh_a = (((i_a // VP) % VA) == j_a).astype(jnp.float32)
    i_p = lax.broadcasted_iota(jnp.int32, (_SBR, VP), 0) + r0
    j_p = lax.broadcasted_iota(jnp.int32, (_SBR, VP), 1)
    oh_p = ((i_p % VP) == j_p).astype(jnp.float32)
    acc = jnp.dot(oh_t, wt_ref[...], preferred_element_type=jnp.float32)
    acc += jnp.dot(oh_a, wa_ref[...], preferred_element_type=jnp.float32)
    acc += jnp.dot(oh_p, wp_ref[...], preferred_element_type=jnp.float32)
    o_ref[...] = acc


def _small_table(wt, wa, wp):
    return pl.pallas_call(
        _small_body,
        grid=(VS // _SBR,),
        in_specs=[
            pl.BlockSpec((VT, D), lambda i: (0, 0)),
            pl.BlockSpec((VA, D), lambda i: (0, 0)),
            pl.BlockSpec((VP, D), lambda i: (0, 0)),
        ],
        out_specs=pl.BlockSpec((_SBR, D), lambda i: (i, 0)),
        out_shape=jax.ShapeDtypeStruct((VS, D), jnp.float32),
    )(wt, wa, wp)


def _sidx_body(t_ref, a_ref, p_ref, o_ref):
    o_ref[...] = t_ref[...] * (VA * VP) + a_ref[...] * VP + p_ref[...]


def _small_index(t, a, p):
    rows = N // D
    shp = jax.ShapeDtypeStruct((rows, D), jnp.int32)
    return pl.pallas_call(_sidx_body, out_shape=shp)(
        t.reshape(rows, D), a.reshape(rows, D), p.reshape(rows, D)
    ).reshape(N)


def _sc_body(qid_hbm, s_hbm, wqp_hbm, wsmall_hbm, out_hbm, *scratch):
    qid_v = scratch[0:NBUF]
    s_v = scratch[NBUF:2 * NBUF]
    rows_v = scratch[2 * NBUF:3 * NBUF]
    sem_qid = scratch[3 * NBUF]
    sem_s = scratch[3 * NBUF + 1]
    sem_g1 = scratch[3 * NBUF + 2]
    sem_g2 = scratch[3 * NBUF + 3]
    sem_st = scratch[3 * NBUF + 4]

    cid = lax.axis_index("c")
    sid = lax.axis_index("s")
    wid = sid * NC + cid
    base = wid * TPW

    def stage_a(g, b):
        off = base + g * C
        pltpu.async_copy(qid_hbm.at[pl.ds(off, C)], qid_v[b], sem_qid[b])
        pltpu.async_copy(s_hbm.at[pl.ds(off, C)], s_v[b], sem_s[b])

    def stage_b(g, b):
        off = base + g * C
        pltpu.make_async_copy(qid_hbm.at[pl.ds(off, C)], qid_v[b], sem_qid[b]).wait()
        pltpu.make_async_copy(s_hbm.at[pl.ds(off, C)], s_v[b], sem_s[b]).wait()
        pltpu.async_copy(wqp_hbm.at[qid_v[b]], rows_v[b], sem_g1[b])

    def stage_c(g, b):
        pltpu.make_async_copy(wqp_hbm.at[qid_v[b]], rows_v[b], sem_g1[b]).wait()
        pltpu.async_copy(wsmall_hbm.at[s_v[b]], rows_v[b], sem_g2[b], add=True)

    def stage_e(g, b):
        off = base + g * C
        pltpu.make_async_copy(wsmall_hbm.at[s_v[b]], rows_v[b], sem_g2[b]).wait()
        pltpu.async_copy(rows_v[b], out_hbm.at[pl.ds(off, C)], sem_st[b])

    def stage_f(g, b):
        off = base + g * C
        pltpu.make_async_copy(rows_v[b], out_hbm.at[pl.ds(off, C)], sem_st[b]).wait()

    for b in range(NBUF):
        g = b
        stage_a(g, b)
        if g >= 1:
            stage_b(g - 1, (b - 1) % NBUF)
        if g >= 2:
            stage_c(g - 2, (b - 2) % NBUF)
        if g >= 3:
            stage_e(g - 3, (b - 3) % NBUF)

    def group_body(i, carry):
        g0 = i * NBUF
        for b in range(NBUF):
            g = g0 + b
            stage_f(g - NBUF, b)
            stage_a(g, b)
            stage_b(g - 1, (b - 1) % NBUF)
            stage_c(g - 2, (b - 2) % NBUF)
            stage_e(g - 3, (b - 3) % NBUF)
        return carry

    lax.fori_loop(1, NCHUNK // NBUF, group_body, 0)

    last = NCHUNK
    for k in range(NBUF):
        g = last + k
        b = g % NBUF
        stage_f(g - NBUF, b)
        if g - 1 < last:
            stage_b(g - 1, (b - 1) % NBUF)
        if g - 2 < last:
            stage_c(g - 2, (b - 2) % NBUF)
        if g - 3 < last:
            stage_e(g - 3, (b - 3) % NBUF)


_sc_embed = functools.partial(
    pl.kernel,
    out_type=jax.ShapeDtypeStruct((N, D), jnp.float32),
    mesh=plsc.VectorSubcoreMesh(core_axis_name="c", subcore_axis_name="s"),
    scratch_types=(
        [pltpu.VMEM((C,), jnp.int32) for _ in range(NBUF)]
        + [pltpu.VMEM((C,), jnp.int32) for _ in range(NBUF)]
        + [pltpu.VMEM((C, D), jnp.float32) for _ in range(NBUF)]
        + [[pltpu.SemaphoreType.DMA for _ in range(NBUF)]
           ] + [[pltpu.SemaphoreType.DMA for _ in range(NBUF)]
           ] + [[pltpu.SemaphoreType.DMA for _ in range(NBUF)]
           ] + [[pltpu.SemaphoreType.DMA for _ in range(NBUF)]
           ] + [[pltpu.SemaphoreType.DMA for _ in range(NBUF)]]
    ),
)(_sc_body)


def kernel(qid_list, input_processed_assessment_list, input_finished_time_list,
           part_list, W_question, W_position, W_assessment, W_finished_time,
           W_part):
    qid = qid_list.reshape(N).astype(jnp.int32)
    a = input_processed_assessment_list.reshape(N).astype(jnp.int32)
    t = input_finished_time_list.reshape(N).astype(jnp.int32)
    p = part_list.reshape(N).astype(jnp.int32)
    wqp = _table_add(W_question, W_position)
    wsmall = _small_table(W_finished_time, W_assessment, W_part)
    s = _small_index(t, a, p)
    out = _sc_embed(qid, s, wqp, wsmall)
    return out.reshape(B, L, D)


# speedup vs baseline: 29.4805x; 1.1763x over previous
import functools

import jax
import jax.numpy as jnp
from jax import lax
from jax.experimental import pallas as pl
from jax.experimental.pallas import tpu as pltpu
from jax.experimental.pallas import tpu_sc as plsc

B, L, D = 4096, 200, 128
VQ, VA, VT, VP = 100000, 4, 301, 8
N = B * L
VS = VT * VA * VP
VS_PAD = 9728

NC, NS = 2, 16
NW = NC * NS
TPW = N // NW
C = 80
NCHUNK = TPW // C
NBUF = 4


def _add_body(a_ref, b_ref, o_ref):
    o_ref[...] = a_ref[...] + b_ref[...]


def _table_add(a, b):
    rows = a.shape[0]
    br = 1000
    return pl.pallas_call(
        _add_body,
        grid=(rows // br,),
        in_specs=[pl.BlockSpec((br, D), lambda i: (i, 0))] * 2,
        out_specs=pl.BlockSpec((br, D), lambda i: (i, 0)),
        out_shape=jax.ShapeDtypeStruct((rows, D), jnp.float32),
    )(a, b)


_SBR = 2432


def _small_body(wt_ref, wa_ref, wp_ref, o_ref):
    r0 = pl.program_id(0) * _SBR
    i_t = lax.broadcasted_iota(jnp.int32, (_SBR, VT), 0) + r0
    j_t = lax.broadcasted_iota(jnp.int32, (_SBR, VT), 1)
    oh_t = ((i_t // (VA * VP)) == j_t).astype(jnp.float32)
    i_a = lax.broadcasted_iota(jnp.int32, (_SBR, VA), 0) + r0
    j_a = lax.broadcasted_iota(jnp.int32, (_SBR, VA), 1)
    oh_a = (((i_a // VP) % VA) == j_a).astype(jnp.float32)
    i_p = lax.broadcasted_iota(jnp.int32, (_SBR, VP), 0) + r0
    j_p = lax.broadcasted_iota(jnp.int32, (_SBR, VP), 1)
    oh_p = ((i_p % VP) == j_p).astype(jnp.float32)
    acc = jnp.dot(oh_t, wt_ref[...], preferred_element_type=jnp.float32)
    acc += jnp.dot(oh_a, wa_ref[...], preferred_element_type=jnp.float32)
    acc += jnp.dot(oh_p, wp_ref[...], preferred_element_type=jnp.float32)
    o_ref[...] = acc


def _small_table(wt, wa, wp):
    return pl.pallas_call(
        _small_body,
        grid=(VS_PAD // _SBR,),
        in_specs=[
            pl.BlockSpec((VT, D), lambda i: (0, 0)),
            pl.BlockSpec((VA, D), lambda i: (0, 0)),
            pl.BlockSpec((VP, D), lambda i: (0, 0)),
        ],
        out_specs=pl.BlockSpec((_SBR, D), lambda i: (i, 0)),
        out_shape=jax.ShapeDtypeStruct((VS_PAD, D), jnp.float32),
    )(wt, wa, wp)


def _sidx_body(t_ref, a_ref, p_ref, o_ref):
    o_ref[...] = t_ref[...] * (VA * VP) + a_ref[...] * VP + p_ref[...]


def _small_index(t, a, p):
    rows = N // D
    shp = jax.ShapeDtypeStruct((rows, D), jnp.int32)
    return pl.pallas_call(_sidx_body, out_shape=shp)(
        t.reshape(rows, D), a.reshape(rows, D), p.reshape(rows, D)
    ).reshape(N)


_VS_PER_SUB = VS_PAD // NS


def _sc_body(qid_hbm, s_hbm, wqp_hbm, wsmall_hbm, out_hbm, *scratch):
    qid_v = scratch[0:NBUF]
    s_v = scratch[NBUF:2 * NBUF]
    rows_v = scratch[2 * NBUF:3 * NBUF]
    wsmall_sp = scratch[3 * NBUF]
    sem_qid = scratch[3 * NBUF + 1]
    sem_s = scratch[3 * NBUF + 2]
    sem_g1 = scratch[3 * NBUF + 3]
    sem_g2 = scratch[3 * NBUF + 4]
    sem_st = scratch[3 * NBUF + 5]

    cid = lax.axis_index("c")
    sid = lax.axis_index("s")
    wid = sid * NC + cid
    base = wid * TPW

    row0 = sid * _VS_PER_SUB
    pltpu.sync_copy(wsmall_hbm.at[pl.ds(row0, _VS_PER_SUB)],
                    wsmall_sp.at[pl.ds(row0, _VS_PER_SUB)])
    plsc.subcore_barrier()

    def stage_a(g, b):
        off = base + g * C
        pltpu.async_copy(qid_hbm.at[pl.ds(off, C)], qid_v[b], sem_qid[b])
        pltpu.async_copy(s_hbm.at[pl.ds(off, C)], s_v[b], sem_s[b])

    def stage_b(g, b):
        off = base + g * C
        pltpu.make_async_copy(qid_hbm.at[pl.ds(off, C)], qid_v[b], sem_qid[b]).wait()
        pltpu.make_async_copy(s_hbm.at[pl.ds(off, C)], s_v[b], sem_s[b]).wait()
        pltpu.async_copy(wqp_hbm.at[qid_v[b]], rows_v[b], sem_g1[b])

    def stage_c(g, b):
        pltpu.make_async_copy(wqp_hbm.at[qid_v[b]], rows_v[b], sem_g1[b]).wait()
        pltpu.async_copy(wsmall_sp.at[s_v[b]], rows_v[b], sem_g2[b], add=True)

    def stage_e(g, b):
        off = base + g * C
        pltpu.make_async_copy(wsmall_sp.at[s_v[b]], rows_v[b], sem_g2[b]).wait()
        pltpu.async_copy(rows_v[b], out_hbm.at[pl.ds(off, C)], sem_st[b])

    def stage_f(g, b):
        off = base + g * C
        pltpu.make_async_copy(rows_v[b], out_hbm.at[pl.ds(off, C)], sem_st[b]).wait()

    for b in range(NBUF):
        g = b
        stage_a(g, b)
        if g >= 1:
            stage_b(g - 1, (b - 1) % NBUF)
        if g >= 2:
            stage_c(g - 2, (b - 2) % NBUF)
        if g >= 3:
            stage_e(g - 3, (b - 3) % NBUF)

    def group_body(i, carry):
        g0 = i * NBUF
        for b in range(NBUF):
            g = g0 + b
            stage_f(g - NBUF, b)
            stage_a(g, b)
            stage_b(g - 1, (b - 1) % NBUF)
            stage_c(g - 2, (b - 2) % NBUF)
            stage_e(g - 3, (b - 3) % NBUF)
        return carry

    lax.fori_loop(1, NCHUNK // NBUF, group_body, 0)

    last = NCHUNK
    for k in range(NBUF):
        g = last + k
        b = g % NBUF
        stage_f(g - NBUF, b)
        if g - 1 < last:
            stage_b(g - 1, (b - 1) % NBUF)
        if g - 2 < last:
            stage_c(g - 2, (b - 2) % NBUF)
        if g - 3 < last:
            stage_e(g - 3, (b - 3) % NBUF)


_sc_embed = functools.partial(
    pl.kernel,
    out_type=jax.ShapeDtypeStruct((N, D), jnp.float32),
    mesh=plsc.VectorSubcoreMesh(core_axis_name="c", subcore_axis_name="s"),
    scratch_types=(
        [pltpu.VMEM((C,), jnp.int32) for _ in range(NBUF)]
        + [pltpu.VMEM((C,), jnp.int32) for _ in range(NBUF)]
        + [pltpu.VMEM((C, D), jnp.float32) for _ in range(NBUF)]
        + [pltpu.VMEM_SHARED((VS_PAD, D), jnp.float32)]
        + [[pltpu.SemaphoreType.DMA for _ in range(NBUF)]
           ] + [[pltpu.SemaphoreType.DMA for _ in range(NBUF)]
           ] + [[pltpu.SemaphoreType.DMA for _ in range(NBUF)]
           ] + [[pltpu.SemaphoreType.DMA for _ in range(NBUF)]
           ] + [[pltpu.SemaphoreType.DMA for _ in range(NBUF)]]
    ),
)(_sc_body)


def kernel(qid_list, input_processed_assessment_list, input_finished_time_list,
           part_list, W_question, W_position, W_assessment, W_finished_time,
           W_part):
    qid = qid_list.reshape(N).astype(jnp.int32)
    a = input_processed_assessment_list.reshape(N).astype(jnp.int32)
    t = input_finished_time_list.reshape(N).astype(jnp.int32)
    p = part_list.reshape(N).astype(jnp.int32)
    wqp = _table_add(W_question, W_position)
    wsmall = _small_table(W_finished_time, W_assessment, W_part)
    s = _small_index(t, a, p)
    out = _sc_embed(qid, s, wqp, wsmall)
    return out.reshape(B, L, D)


# speedup vs baseline: 30.5040x; 1.0347x over previous
import functools

import jax
import jax.numpy as jnp
from jax import lax
from jax.experimental import pallas as pl
from jax.experimental.pallas import tpu as pltpu
from jax.experimental.pallas import tpu_sc as plsc

B, L, D = 4096, 200, 128
VQ, VA, VT, VP = 100000, 4, 301, 8
N = B * L
VS = VT * VA * VP
VS_PAD = 9728

NC, NS = 2, 16
NW = NC * NS
TPW = N // NW
C = 64
NCHUNK = TPW // C
NBUF = 5


def _add_body(a_ref, b_ref, o_ref):
    o_ref[...] = a_ref[...] + b_ref[...]


def _table_add(a, b):
    rows = a.shape[0]
    br = 1000
    return pl.pallas_call(
        _add_body,
        grid=(rows // br,),
        in_specs=[pl.BlockSpec((br, D), lambda i: (i, 0))] * 2,
        out_specs=pl.BlockSpec((br, D), lambda i: (i, 0)),
        out_shape=jax.ShapeDtypeStruct((rows, D), jnp.float32),
    )(a, b)


_SBR = 2432


def _small_body(wt_ref, wa_ref, wp_ref, o_ref):
    r0 = pl.program_id(0) * _SBR
    i_t = lax.broadcasted_iota(jnp.int32, (_SBR, VT), 0) + r0
    j_t = lax.broadcasted_iota(jnp.int32, (_SBR, VT), 1)
    oh_t = ((i_t // (VA * VP)) == j_t).astype(jnp.float32)
    i_a = lax.broadcasted_iota(jnp.int32, (_SBR, VA), 0) + r0
    j_a = lax.broadcasted_iota(jnp.int32, (_SBR, VA), 1)
    oh_a = (((i_a // VP) % VA) == j_a).astype(jnp.float32)
    i_p = lax.broadcasted_iota(jnp.int32, (_SBR, VP), 0) + r0
    j_p = lax.broadcasted_iota(jnp.int32, (_SBR, VP), 1)
    oh_p = ((i_p % VP) == j_p).astype(jnp.float32)
    acc = jnp.dot(oh_t, wt_ref[...], preferred_element_type=jnp.float32)
    acc += jnp.dot(oh_a, wa_ref[...], preferred_element_type=jnp.float32)
    acc += jnp.dot(oh_p, wp_ref[...], preferred_element_type=jnp.float32)
    o_ref[...] = acc


def _small_table(wt, wa, wp):
    return pl.pallas_call(
        _small_body,
        grid=(VS_PAD // _SBR,),
        in_specs=[
            pl.BlockSpec((VT, D), lambda i: (0, 0)),
            pl.BlockSpec((VA, D), lambda i: (0, 0)),
            pl.BlockSpec((VP, D), lambda i: (0, 0)),
        ],
        out_specs=pl.BlockSpec((_SBR, D), lambda i: (i, 0)),
        out_shape=jax.ShapeDtypeStruct((VS_PAD, D), jnp.float32),
    )(wt, wa, wp)


def _sidx_body(t_ref, a_ref, p_ref, o_ref):
    o_ref[...] = t_ref[...] * (VA * VP) + a_ref[...] * VP + p_ref[...]


def _small_index(t, a, p):
    rows = N // D
    shp = jax.ShapeDtypeStruct((rows, D), jnp.int32)
    return pl.pallas_call(_sidx_body, out_shape=shp)(
        t.reshape(rows, D), a.reshape(rows, D), p.reshape(rows, D)
    ).reshape(N)


_VS_PER_SUB = VS_PAD // NS


def _sc_body(qid_hbm, s_hbm, wqp_hbm, wsmall_hbm, out_hbm, *scratch):
    qid_v = scratch[0:NBUF]
    s_v = scratch[NBUF:2 * NBUF]
    rows_v = scratch[2 * NBUF:3 * NBUF]
    wsmall_sp = scratch[3 * NBUF]
    sem_qid = scratch[3 * NBUF + 1]
    sem_s = scratch[3 * NBUF + 2]
    sem_g1 = scratch[3 * NBUF + 3]
    sem_g2 = scratch[3 * NBUF + 4]
    sem_st = scratch[3 * NBUF + 5]

    cid = lax.axis_index("c")
    sid = lax.axis_index("s")
    wid = sid * NC + cid
    base = wid * TPW

    row0 = sid * _VS_PER_SUB
    pltpu.sync_copy(wsmall_hbm.at[pl.ds(row0, _VS_PER_SUB)],
                    wsmall_sp.at[pl.ds(row0, _VS_PER_SUB)])
    plsc.subcore_barrier()

    def stage_a(g, b):
        off = base + g * C
        pltpu.async_copy(qid_hbm.at[pl.ds(off, C)], qid_v[b], sem_qid[b])
        pltpu.async_copy(s_hbm.at[pl.ds(off, C)], s_v[b], sem_s[b])

    def stage_b(g, b):
        off = base + g * C
        pltpu.make_async_copy(qid_hbm.at[pl.ds(off, C)], qid_v[b], sem_qid[b]).wait()
        pltpu.make_async_copy(s_hbm.at[pl.ds(off, C)], s_v[b], sem_s[b]).wait()
        pltpu.async_copy(wqp_hbm.at[qid_v[b]], rows_v[b], sem_g1[b])

    def stage_c(g, b):
        pltpu.make_async_copy(wqp_hbm.at[qid_v[b]], rows_v[b], sem_g1[b]).wait()
        pltpu.async_copy(wsmall_sp.at[s_v[b]], rows_v[b], sem_g2[b], add=True)

    def stage_e(g, b):
        off = base + g * C
        pltpu.make_async_copy(wsmall_sp.at[s_v[b]], rows_v[b], sem_g2[b]).wait()
        pltpu.async_copy(rows_v[b], out_hbm.at[pl.ds(off, C)], sem_st[b])

    def stage_f(g, b):
        off = base + g * C
        pltpu.make_async_copy(rows_v[b], out_hbm.at[pl.ds(off, C)], sem_st[b]).wait()

    for b in range(NBUF):
        g = b
        stage_a(g, b)
        if g >= 1:
            stage_b(g - 1, (b - 1) % NBUF)
        if g >= 2:
            stage_c(g - 2, (b - 2) % NBUF)
        if g >= 3:
            stage_e(g - 3, (b - 3) % NBUF)

    def group_body(i, carry):
        g0 = i * NBUF
        for b in range(NBUF):
            g = g0 + b
            stage_f(g - NBUF, b)
            stage_a(g, b)
            stage_b(g - 1, (b - 1) % NBUF)
            stage_c(g - 2, (b - 2) % NBUF)
            stage_e(g - 3, (b - 3) % NBUF)
        return carry

    lax.fori_loop(1, NCHUNK // NBUF, group_body, 0)

    last = NCHUNK
    for k in range(NBUF):
        g = last + k
        b = g % NBUF
        stage_f(g - NBUF, b)
        if g - 1 < last:
            stage_b(g - 1, (b - 1) % NBUF)
        if g - 2 < last:
            stage_c(g - 2, (b - 2) % NBUF)
        if g - 3 < last:
            stage_e(g - 3, (b - 3) % NBUF)


_sc_embed = functools.partial(
    pl.kernel,
    out_type=jax.ShapeDtypeStruct((N, D), jnp.float32),
    mesh=plsc.VectorSubcoreMesh(core_axis_name="c", subcore_axis_name="s"),
    scratch_types=(
        [pltpu.VMEM((C,), jnp.int32) for _ in range(NBUF)]
        + [pltpu.VMEM((C,), jnp.int32) for _ in range(NBUF)]
        + [pltpu.VMEM((C, D), jnp.float32) for _ in range(NBUF)]
        + [pltpu.VMEM_SHARED((VS_PAD, D), jnp.float32)]
        + [[pltpu.SemaphoreType.DMA for _ in range(NBUF)]
           ] + [[pltpu.SemaphoreType.DMA for _ in range(NBUF)]
           ] + [[pltpu.SemaphoreType.DMA for _ in range(NBUF)]
           ] + [[pltpu.SemaphoreType.DMA for _ in range(NBUF)]
           ] + [[pltpu.SemaphoreType.DMA for _ in range(NBUF)]]
    ),
)(_sc_body)


def kernel(qid_list, input_processed_assessment_list, input_finished_time_list,
           part_list, W_question, W_position, W_assessment, W_finished_time,
           W_part):
    qid = qid_list.reshape(N).astype(jnp.int32)
    a = input_processed_assessment_list.reshape(N).astype(jnp.int32)
    t = input_finished_time_list.reshape(N).astype(jnp.int32)
    p = part_list.reshape(N).astype(jnp.int32)
    wqp = _table_add(W_question, W_position)
    wsmall = _small_table(W_finished_time, W_assessment, W_part)
    s = _small_index(t, a, p)
    out = _sc_embed(qid, s, wqp, wsmall)
    return out.reshape(B, L, D)
